# baseline (speedup 1.0000x reference)
import jax
import jax.numpy as jnp
from jax.experimental import pallas as pl
from jax.experimental.pallas import tpu as pltpu

_BM = 200


def _fused_kernel(x_ref, adj_ref, w1_ref, w2_ref,
                  h1_ref, adj_out_ref, logits_ref,
                  s1_scr, hw2_scr):
    s = pl.program_id(0)
    i = pl.program_id(1)

    @pl.when((s == 0) & (i == 0))
    def _():
        s1_scr[...] = jnp.dot(x_ref[...], w1_ref[...],
                              preferred_element_type=jnp.float32)

    a = adj_ref[...]

    @pl.when(s == 0)
    def _():
        h1 = jnp.maximum(
            jnp.dot(a, s1_scr[...], preferred_element_type=jnp.float32), 0.0)
        h1_ref[...] = h1
        hw2_scr[pl.ds(i * _BM, _BM), :] = jnp.dot(
            h1, w2_ref[...], preferred_element_type=jnp.float32)

    @pl.when(s == 1)
    def _():
        adj_out_ref[...] = a
        x2 = jnp.maximum(
            jnp.dot(a, hw2_scr[...], preferred_element_type=jnp.float32), 0.0)
        m = jnp.max(x2, axis=1, keepdims=True)
        e = jnp.exp(x2 - m)
        logits_ref[...] = (x2 - m) - jnp.log(
            jnp.sum(e, axis=1, keepdims=True))


def kernel(x, adj, W1, W2):
    n, nfeat = x.shape
    nhid = W1.shape[1]
    nclass = W2.shape[1]
    ns = n // _BM

    full = lambda s, i: (0, 0)
    every = lambda s, i: (i, 0)
    ph0 = lambda s, i: (jnp.where(s == 0, i, ns - 1), 0)
    ph1 = lambda s, i: (jnp.where(s == 1, i, 0), 0)

    h1, adj_out, logits = pl.pallas_call(
        _fused_kernel,
        grid=(2, ns),
        in_specs=[
            pl.BlockSpec((n, nfeat), full),
            pl.BlockSpec((_BM, n), every),
            pl.BlockSpec((nfeat, nhid), full),
            pl.BlockSpec((nhid, nclass), full),
        ],
        out_specs=[
            pl.BlockSpec((_BM, nhid), ph0),
            pl.BlockSpec((_BM, n), ph1),
            pl.BlockSpec((_BM, nclass), ph1),
        ],
        out_shape=[
            jax.ShapeDtypeStruct((n, nhid), jnp.float32),
            jax.ShapeDtypeStruct((n, n), jnp.float32),
            jax.ShapeDtypeStruct((n, nclass), jnp.float32),
        ],
        scratch_shapes=[
            pltpu.VMEM((n, nhid), jnp.float32),
            pltpu.VMEM((n, nclass), jnp.float32),
        ],
        compiler_params=pltpu.CompilerParams(
            dimension_semantics=("arbitrary", "arbitrary"),
        ),
    )(x, adj, W1, W2)
    return (logits, h1, adj_out)
